# Initial kernel scaffold; baseline (speedup 1.0000x reference)
import jax
import jax.numpy as jnp
from jax.experimental import pallas as pl


def kernel(voxel_features, vertices, pad_img_shape):
    raise NotImplementedError("write your pallas kernel here")


# speedup vs baseline: 1.7245x; 1.7245x over previous
import functools

import jax
import jax.numpy as jnp
from jax import lax
from jax.experimental import pallas as pl
from jax.experimental.pallas import tpu as pltpu
from jax.experimental.pallas import tpu_sc as plsc

B = 4
C = 32
D = H = W = 64
N = 100000
NTOT = B * N

NC = 2
NS = 16
NW = NC * NS
L = 16

CV = 128
NCH = 98
NV = CV * NCH
NTOTP = NW * NV

DHW = D * H * W

_CORNER_OFFS = (0, W, H * W, H * W + W, 1, W + 1, H * W + 1, H * W + W + 1)


def _axis01(v, extent):
    t = (v + 1.0) * 0.5 * (extent - 1)
    t = jnp.clip(t, 0.0, float(extent - 1))
    i0 = jnp.minimum(t.astype(jnp.int32), extent - 2)
    frac = t - i0.astype(jnp.float32)
    return i0, frac


def _sc_body(table, xs, ys, zs, out, idx_v, rows_v, wgt_v, cx_v, cy_v, cz_v,
             outb_v, sem_g, sem_c):
    wid = lax.axis_index("s") * NC + lax.axis_index("c")
    vbase = wid * NV

    def chunk_body(ci, _):
        base = vbase + ci * CV
        cpy = [
            pltpu.async_copy(xs.at[pl.ds(base, CV)], cx_v, sem_c),
            pltpu.async_copy(ys.at[pl.ds(base, CV)], cy_v, sem_c),
            pltpu.async_copy(zs.at[pl.ds(base, CV)], cz_v, sem_c),
        ]
        for c in cpy:
            c.wait()

        for g in range(CV // L):
            sl = pl.ds(g * L, L)
            xv = cx_v[sl]
            yv = cy_v[sl]
            zv = cz_v[sl]
            gid = base + g * L + lax.iota(jnp.int32, L)
            one = jnp.ones((L,), jnp.int32)
            zero = jnp.zeros((L,), jnp.int32)
            bid = (jnp.where(gid >= N, one, zero)
                   + jnp.where(gid >= 2 * N, one, zero)
                   + jnp.where(gid >= 3 * N, one, zero))
            x0, wx = _axis01(xv, W)
            y0, wy = _axis01(yv, H)
            z0, wz = _axis01(zv, D)
            rbase = (bid << 18) + (z0 << 12) + (y0 << 6) + x0
            for k in range(8):
                idx_v[k, sl] = rbase + _CORNER_OFFS[k]
            wy1 = wy
            wy0 = 1.0 - wy
            wz1 = wz
            wz0 = 1.0 - wz
            wgt_v[pl.ds(0 * CV + g * L, L)] = wx
            wgt_v[pl.ds(1 * CV + g * L, L)] = wz0 * wy0
            wgt_v[pl.ds(2 * CV + g * L, L)] = wz0 * wy1
            wgt_v[pl.ds(3 * CV + g * L, L)] = wz1 * wy0
            wgt_v[pl.ds(4 * CV + g * L, L)] = wz1 * wy1

        gcopies = [
            pltpu.async_copy(table.at[idx_v.at[k]], rows_v.at[k], sem_g)
            for k in range(8)
        ]
        for c in gcopies:
            c.wait()

        def lerp_group(g, _):
            wxv = wgt_v[pl.ds(0 * CV + g * L, L)]
            w00v = wgt_v[pl.ds(1 * CV + g * L, L)]
            w01v = wgt_v[pl.ds(2 * CV + g * L, L)]
            w10v = wgt_v[pl.ds(3 * CV + g * L, L)]
            w11v = wgt_v[pl.ds(4 * CV + g * L, L)]
            for lane in range(L):
                i = g * L + lane
                wx = wxv[lane]
                w00 = w00v[lane]
                w01 = w01v[lane]
                w10 = w10v[lane]
                w11 = w11v[lane]
                for h in range(C // L):
                    hs = pl.ds(h * L, L)
                    a = (w00 * rows_v[0, i, hs] + w01 * rows_v[1, i, hs]
                         + w10 * rows_v[2, i, hs] + w11 * rows_v[3, i, hs])
                    b = (w00 * rows_v[4, i, hs] + w01 * rows_v[5, i, hs]
                         + w10 * rows_v[6, i, hs] + w11 * rows_v[7, i, hs])
                    outb_v[i, hs] = a + wx * (b - a)
            return 0

        lax.fori_loop(0, CV // L, lerp_group, 0)
        pltpu.sync_copy(outb_v, out.at[pl.ds(base, CV)])
        return 0

    lax.fori_loop(0, NCH, chunk_body, 0)


@jax.jit
def _run(table, xs, ys, zs):
    mesh = plsc.VectorSubcoreMesh(core_axis_name="c", subcore_axis_name="s")
    return pl.kernel(
        _sc_body,
        out_type=jax.ShapeDtypeStruct((NTOTP, C), jnp.float32),
        mesh=mesh,
        compiler_params=pltpu.CompilerParams(use_tc_tiling_on_sc=False),
        scratch_types=[
            pltpu.VMEM((8, CV), jnp.int32),
            pltpu.VMEM((8, CV, C), jnp.float32),
            pltpu.VMEM((5 * CV,), jnp.float32),
            pltpu.VMEM((CV,), jnp.float32),
            pltpu.VMEM((CV,), jnp.float32),
            pltpu.VMEM((CV,), jnp.float32),
            pltpu.VMEM((CV, C), jnp.float32),
            pltpu.SemaphoreType.DMA,
            pltpu.SemaphoreType.DMA,
        ],
    )(table, xs, ys, zs)


def kernel(voxel_features, vertices, pad_img_shape):
    del pad_img_shape
    table = jnp.transpose(voxel_features, (0, 2, 3, 4, 1)).reshape(B * DHW, C)
    verts = vertices.reshape(NTOT, 3)
    verts = jnp.pad(verts, ((0, NTOTP - NTOT), (0, 0)))
    xs = verts[:, 0]
    ys = verts[:, 1]
    zs = verts[:, 2]
    out = _run(table, xs, ys, zs)
    return out[:NTOT].reshape(B, N, C)
